# baseline (speedup 1.0000x reference)
import functools

import jax
import jax.numpy as jnp
from jax import lax
from jax.experimental import pallas as pl
from jax.experimental.pallas import tpu as pltpu
from jax.experimental.pallas import tpu_sc as plsc

N_U = 10000
D = 128
E_REL = 320000
CHUNK = 80
NS = 16
NC = 2
CH_PER_CORE = E_REL // CHUNK
CH_MAIN = CH_PER_CORE // NS
NROW = 4
NIDX = 8
R_MAIN = 624
R_TAIL = N_U - R_MAIN * NS


def _sc_aggregate(x_stack, src_all, dst_all):
    mesh = plsc.VectorSubcoreMesh(core_axis_name="c", subcore_axis_name="s")

    @functools.partial(
        pl.kernel,
        out_type=(
            jax.ShapeDtypeStruct((NC * N_U, D), jnp.float32),
            jax.ShapeDtypeStruct((NC * N_U,), jnp.float32),
        ),
        mesh=mesh,
        scratch_types=[
            [pltpu.VMEM((8 * CHUNK,), jnp.int32) for _ in range(2)],
            [pltpu.VMEM((CHUNK,), jnp.int32) for _ in range(NIDX)],
            [pltpu.VMEM((CHUNK, D), jnp.float32) for _ in range(NROW)],
            pltpu.VMEM((CHUNK,), jnp.float32),
            pltpu.VMEM((R_MAIN,), jnp.float32),
            pltpu.VMEM_SHARED((N_U, D), jnp.float32),
            pltpu.VMEM_SHARED((N_U,), jnp.float32),
            [pltpu.SemaphoreType.DMA for _ in range(2)],
            [pltpu.SemaphoreType.DMA for _ in range(NIDX)],
            [pltpu.SemaphoreType.DMA for _ in range(NROW)],
            [pltpu.SemaphoreType.DMA for _ in range(NROW)],
        ],
    )
    def k(x_hbm, src_hbm, dst_hbm, agg_hbm, deg_hbm,
          srcg_v, dst_v, rows_v, ones_v, zdeg_v, acc_sh, deg_sh,
          sgi, si, sg, ss):
        c = lax.axis_index("c")
        s = lax.axis_index("s")

        def init_ones(i, carry):
            ones_v[pl.ds(i * 16, 16)] = jnp.full((16,), 1.0, jnp.float32)
            return carry
        lax.fori_loop(0, CHUNK // 16, init_ones, 0)

        def init_zdeg(i, carry):
            zdeg_v[pl.ds(i * 16, 16)] = jnp.zeros((16,), jnp.float32)
            return carry
        lax.fori_loop(0, R_MAIN // 16, init_zdeg, 0)

        def init_rows(i, carry):
            rows_v[0][i // 8, pl.ds((i % 8) * 16, 16)] = jnp.zeros((16,), jnp.float32)
            return carry
        lax.fori_loop(0, CHUNK * 8, init_rows, 0)

        row0 = s * R_MAIN
        for kk in range(R_MAIN // CHUNK):
            pltpu.sync_copy(rows_v[0], acc_sh.at[pl.ds(row0 + kk * CHUNK, CHUNK)])
        rem = R_MAIN % CHUNK
        pltpu.sync_copy(rows_v[0].at[pl.ds(0, rem)],
                        acc_sh.at[pl.ds(row0 + R_MAIN - rem, rem)])
        pltpu.sync_copy(zdeg_v, deg_sh.at[pl.ds(row0, R_MAIN)])

        @pl.when(s == NS - 1)
        def _():
            pltpu.sync_copy(rows_v[0].at[pl.ds(0, R_TAIL)],
                            acc_sh.at[pl.ds(N_U - R_TAIL, R_TAIL)])
            pltpu.sync_copy(zdeg_v.at[pl.ds(0, R_TAIL)],
                            deg_sh.at[pl.ds(N_U - R_TAIL, R_TAIL)])

        plsc.subcore_barrier()

        base_ch = c * CH_PER_CORE + s * CH_MAIN

        def ebase(g):
            return (base_ch + g) * CHUNK

        def issue_grp(gg, nch, b):
            pltpu.async_copy(
                src_hbm.at[pl.ds(ebase(8 * gg), nch * CHUNK)],
                srcg_v[b].at[pl.ds(0, nch * CHUNK)], sgi[b])

        def wait_grp(gg, nch, b):
            pltpu.make_async_copy(
                src_hbm.at[pl.ds(ebase(8 * gg), nch * CHUNK)],
                srcg_v[b].at[pl.ds(0, nch * CHUNK)], sgi[b]).wait()

        def issue_idx(g, r):
            pltpu.async_copy(dst_hbm.at[pl.ds(ebase(g), CHUNK)], dst_v[r], si[r])

        def wait_idx(g, r):
            pltpu.make_async_copy(dst_hbm.at[pl.ds(ebase(g), CHUNK)],
                                  dst_v[r], si[r]).wait()

        def issue_gather(j8, b, p):
            pltpu.async_copy(x_hbm.at[srcg_v[b].at[pl.ds(j8 * CHUNK, CHUNK)]],
                             rows_v[p], sg[p])

        def wait_gather(j8, b, p):
            pltpu.make_async_copy(
                x_hbm.at[srcg_v[b].at[pl.ds(j8 * CHUNK, CHUNK)]],
                rows_v[p], sg[p]).wait()

        def issue_scatter(r, p):
            pltpu.async_copy(rows_v[p], acc_sh.at[dst_v[r]], ss[p], add=True)
            pltpu.async_copy(ones_v, deg_sh.at[dst_v[r]], ss[p], add=True)

        def wait_scatter(r, p):
            pltpu.make_async_copy(rows_v[p], acc_sh.at[dst_v[r]], ss[p]).wait()
            pltpu.make_async_copy(ones_v, deg_sh.at[dst_v[r]], ss[p]).wait()

        def step(g, j, b, bn, drain_sc=True, pf_idx=True, issue_g=True):
            p = j % NROW
            r = j
            p2 = (j + 2) % NROW
            r2f = (j + 6) % NIDX
            r2 = (j + 2) % NIDX
            r5 = (j + 5) % NIDX
            wait_gather(j, b, p)
            issue_scatter(r, p)
            if drain_sc:
                wait_scatter(r2f, p2)
            if issue_g:
                wait_idx(g + 2, r2)
                issue_gather((j + 2) % 8, bn, p2)
            if pf_idx:
                issue_idx(g + 5, r5)

        NGRP = CH_MAIN // 8
        LAST_N = CH_MAIN - 8 * NGRP

        issue_grp(0, 8, 0)
        issue_grp(1, 8, 1)
        for g in range(5):
            issue_idx(g, g)
        wait_grp(0, 8, 0)
        wait_idx(0, 0)
        issue_gather(0, 0, 0)
        wait_idx(1, 1)
        issue_gather(1, 0, 1)

        step(0, 0, 0, 0, drain_sc=False)
        step(1, 1, 0, 0, drain_sc=False)
        for j in range(2, 8):
            if j == 6:
                wait_grp(1, 8, 1)
            step(j, j, 0, 1 if j >= 6 else 0)

        def body(i, carry):
            for half in range(2):
                G = 1 + 2 * i + half
                b = (1 + half) % 2
                g0 = 8 * G
                issue_grp(G + 1, 8, 1 - b)
                for j in range(8):
                    if j == 6:
                        wait_grp(G + 1, 8, 1 - b)
                    step(g0 + j, j, b, (1 - b) if j >= 6 else b)
            return carry
        lax.fori_loop(0, (NGRP - 3) // 2, body, 0)

        issue_grp(29 + 1, 8, 0)
        for j in range(8):
            if j == 6:
                wait_grp(30, 8, 0)
            step(8 * 29 + j, j, 1, 0 if j >= 6 else 1)

        issue_grp(31, LAST_N, 1)
        for j in range(8):
            if j == 6:
                wait_grp(31, LAST_N, 1)
            step(8 * 30 + j, j, 0, 1 if j >= 6 else 0,
                 pf_idx=(8 * 30 + j + 5 < CH_MAIN))

        step(8 * 31 + 0, 0, 1, 1, pf_idx=False, issue_g=False)
        step(8 * 31 + 1, 1, 1, 1, pf_idx=False, issue_g=False)
        wait_scatter((CH_MAIN - 2) % NIDX, (CH_MAIN - 2) % NROW)
        wait_scatter((CH_MAIN - 1) % NIDX, (CH_MAIN - 1) % NROW)

        plsc.subcore_barrier()

        out0 = c * N_U + row0
        pltpu.sync_copy(acc_sh.at[pl.ds(row0, R_MAIN)],
                        agg_hbm.at[pl.ds(out0, R_MAIN)])
        pltpu.sync_copy(deg_sh.at[pl.ds(row0, R_MAIN)], zdeg_v)
        pltpu.sync_copy(zdeg_v, deg_hbm.at[pl.ds(out0, R_MAIN)])

        @pl.when(s == NS - 1)
        def _():
            pltpu.sync_copy(acc_sh.at[pl.ds(N_U - R_TAIL, R_TAIL)],
                            agg_hbm.at[pl.ds(c * N_U + N_U - R_TAIL, R_TAIL)])
            pltpu.sync_copy(deg_sh.at[pl.ds(N_U - R_TAIL, R_TAIL)],
                            ones_v.at[pl.ds(0, R_TAIL)])
            pltpu.sync_copy(ones_v.at[pl.ds(0, R_TAIL)],
                            deg_hbm.at[pl.ds(c * N_U + N_U - R_TAIL, R_TAIL)])

    return k(x_stack, src_all, dst_all)


def _tc_finish(aggs, degs, w_f, w_b):
    BR = 1000
    nblk = N_U // BR
    degs2 = degs.reshape(NC * N_U, 1)

    def body(a0_ref, a1_ref, d0_ref, d1_ref, w0_ref, w1_ref, o_ref):
        d0 = jnp.maximum(d0_ref[...], 1.0)
        d1 = jnp.maximum(d1_ref[...], 1.0)
        a0 = a0_ref[...] / d0
        a1 = a1_ref[...] / d1
        o_ref[...] = (
            jnp.dot(a0, w0_ref[...], preferred_element_type=jnp.float32)
            + jnp.dot(a1, w1_ref[...], preferred_element_type=jnp.float32)
        )

    return pl.pallas_call(
        body,
        grid=(nblk,),
        in_specs=[
            pl.BlockSpec((BR, D), lambda i: (i, 0)),
            pl.BlockSpec((BR, D), lambda i: (i + nblk, 0)),
            pl.BlockSpec((BR, 1), lambda i: (i, 0)),
            pl.BlockSpec((BR, 1), lambda i: (i + nblk, 0)),
            pl.BlockSpec((D, D), lambda i: (0, 0)),
            pl.BlockSpec((D, D), lambda i: (0, 0)),
        ],
        out_specs=pl.BlockSpec((BR, D), lambda i: (i, 0)),
        out_shape=jax.ShapeDtypeStruct((N_U, D), jnp.float32),
    )(aggs, aggs, degs2, degs2, w_f, w_b)


def kernel(x_user, x_item, edge_index_follows, edge_index_bought,
           W_follows, W_bought):
    src_f = edge_index_follows[0].astype(jnp.int32)
    dst_f = edge_index_follows[1].astype(jnp.int32)
    src_b = edge_index_bought[0].astype(jnp.int32) + N_U
    dst_b = edge_index_bought[1].astype(jnp.int32)
    x_stack = jnp.concatenate([x_user, x_item], axis=0)
    src_all = jnp.concatenate([src_f, src_b])
    dst_all = jnp.concatenate([dst_f, dst_b])
    aggs, degs = _sc_aggregate(x_stack, src_all, dst_all)
    return _tc_finish(aggs, degs, W_follows, W_bought)


# speedup vs baseline: 1.0090x; 1.0090x over previous
import functools

import jax
import jax.numpy as jnp
from jax import lax
from jax.experimental import pallas as pl
from jax.experimental.pallas import tpu as pltpu
from jax.experimental.pallas import tpu_sc as plsc

N_U = 10000
D = 128
E_REL = 320000
CHUNK = 80
NS = 16
NC = 2
CH_PER_CORE = E_REL // CHUNK
CH_MAIN = CH_PER_CORE // NS
NROW = 4
NIDX = 8
R_MAIN = 624
R_TAIL = N_U - R_MAIN * NS


def _sc_aggregate(x_stack, src_all, dst_all):
    mesh = plsc.VectorSubcoreMesh(core_axis_name="c", subcore_axis_name="s")

    @functools.partial(
        pl.kernel,
        out_type=(
            jax.ShapeDtypeStruct((NC * N_U, D), jnp.float32),
            jax.ShapeDtypeStruct((NC * N_U,), jnp.float32),
        ),
        mesh=mesh,
        scratch_types=[
            [pltpu.VMEM((CHUNK,), jnp.int32) for _ in range(NIDX)],
            [pltpu.VMEM((CHUNK,), jnp.int32) for _ in range(NIDX)],
            [pltpu.VMEM((CHUNK, D), jnp.float32) for _ in range(NROW)],
            pltpu.VMEM((CHUNK,), jnp.float32),
            pltpu.VMEM((R_MAIN,), jnp.float32),
            pltpu.VMEM_SHARED((N_U, D), jnp.float32),
            pltpu.VMEM_SHARED((N_U,), jnp.float32),
            [pltpu.SemaphoreType.DMA for _ in range(NIDX)],
            [pltpu.SemaphoreType.DMA for _ in range(NROW)],
            [pltpu.SemaphoreType.DMA for _ in range(NROW)],
        ],
    )
    def k(x_hbm, src_hbm, dst_hbm, agg_hbm, deg_hbm,
          src_v, dst_v, rows_v, ones_v, zdeg_v, acc_sh, deg_sh,
          si, sg, ss):
        c = lax.axis_index("c")
        s = lax.axis_index("s")

        def init_ones(i, carry):
            ones_v[pl.ds(i * 16, 16)] = jnp.full((16,), 1.0, jnp.float32)
            return carry
        lax.fori_loop(0, CHUNK // 16, init_ones, 0)

        def init_zdeg(i, carry):
            zdeg_v[pl.ds(i * 16, 16)] = jnp.zeros((16,), jnp.float32)
            return carry
        lax.fori_loop(0, R_MAIN // 16, init_zdeg, 0)

        def init_rows(i, carry):
            rows_v[0][i // 8, pl.ds((i % 8) * 16, 16)] = jnp.zeros((16,), jnp.float32)
            return carry
        lax.fori_loop(0, CHUNK * 8, init_rows, 0)

        row0 = s * R_MAIN
        for kk in range(R_MAIN // CHUNK):
            pltpu.sync_copy(rows_v[0], acc_sh.at[pl.ds(row0 + kk * CHUNK, CHUNK)])
        rem = R_MAIN % CHUNK
        pltpu.sync_copy(rows_v[0].at[pl.ds(0, rem)],
                        acc_sh.at[pl.ds(row0 + R_MAIN - rem, rem)])
        pltpu.sync_copy(zdeg_v, deg_sh.at[pl.ds(row0, R_MAIN)])

        @pl.when(s == NS - 1)
        def _():
            pltpu.sync_copy(rows_v[0].at[pl.ds(0, R_TAIL)],
                            acc_sh.at[pl.ds(N_U - R_TAIL, R_TAIL)])
            pltpu.sync_copy(zdeg_v.at[pl.ds(0, R_TAIL)],
                            deg_sh.at[pl.ds(N_U - R_TAIL, R_TAIL)])

        plsc.subcore_barrier()

        base_ch = c * CH_PER_CORE + s * CH_MAIN

        def ebase(g):
            return (base_ch + g) * CHUNK

        def issue_idx(g, r):
            pltpu.async_copy(src_hbm.at[pl.ds(ebase(g), CHUNK)], src_v[r], si[r])
            pltpu.async_copy(dst_hbm.at[pl.ds(ebase(g), CHUNK)], dst_v[r], si[r])

        def wait_idx(g, r):
            pltpu.make_async_copy(src_hbm.at[pl.ds(ebase(g), CHUNK)],
                                  src_v[r], si[r]).wait()
            pltpu.make_async_copy(dst_hbm.at[pl.ds(ebase(g), CHUNK)],
                                  dst_v[r], si[r]).wait()

        def issue_gather(r, p):
            pltpu.async_copy(x_hbm.at[src_v[r]], rows_v[p], sg[p])

        def wait_gather(r, p):
            pltpu.make_async_copy(x_hbm.at[src_v[r]], rows_v[p], sg[p]).wait()

        def issue_scatter(r, p):
            pltpu.async_copy(rows_v[p], acc_sh.at[dst_v[r]], ss[p], add=True)
            pltpu.async_copy(ones_v, deg_sh.at[dst_v[r]], ss[p], add=True)

        def wait_scatter(r, p):
            pltpu.make_async_copy(rows_v[p], acc_sh.at[dst_v[r]], ss[p]).wait()
            pltpu.make_async_copy(ones_v, deg_sh.at[dst_v[r]], ss[p]).wait()

        def step(g, gm4, gm8, drain_sc=True, pf_idx=True, issue_g=True):
            p, r = gm4, gm8
            p2 = (gm4 + 2) % NROW
            r2f = (gm8 + 6) % NIDX
            r2 = (gm8 + 2) % NIDX
            r5 = (gm8 + 5) % NIDX
            wait_gather(r, p)
            issue_scatter(r, p)
            if drain_sc:
                wait_scatter(r2f, p2)
            if issue_g:
                wait_idx(g + 2, r2)
                issue_gather(r2, p2)
            if pf_idx:
                issue_idx(g + 5, r5)

        for g in range(5):
            issue_idx(g, g)
        wait_idx(0, 0)
        issue_gather(0, 0)
        wait_idx(1, 1)
        issue_gather(1, 1)
        step(0, 0, 0, drain_sc=False)
        step(1, 1, 1, drain_sc=False)

        def body(i, carry):
            g0 = 2 + i * 8
            for j in range(8):
                step(g0 + j, (2 + j) % NROW, (2 + j) % NIDX)
            return carry
        lax.fori_loop(0, (CH_MAIN - 10) // 8, body, 0)

        for g in range(CH_MAIN - 8, CH_MAIN):
            step(g, g % NROW, g % NIDX,
                 pf_idx=(g + 5 < CH_MAIN), issue_g=(g + 2 < CH_MAIN))
        wait_scatter((CH_MAIN - 2) % NIDX, (CH_MAIN - 2) % NROW)
        wait_scatter((CH_MAIN - 1) % NIDX, (CH_MAIN - 1) % NROW)

        plsc.subcore_barrier()

        out0 = c * N_U + row0
        pltpu.sync_copy(acc_sh.at[pl.ds(row0, R_MAIN)],
                        agg_hbm.at[pl.ds(out0, R_MAIN)])
        pltpu.sync_copy(deg_sh.at[pl.ds(row0, R_MAIN)], zdeg_v)
        pltpu.sync_copy(zdeg_v, deg_hbm.at[pl.ds(out0, R_MAIN)])

        @pl.when(s == NS - 1)
        def _():
            pltpu.sync_copy(acc_sh.at[pl.ds(N_U - R_TAIL, R_TAIL)],
                            agg_hbm.at[pl.ds(c * N_U + N_U - R_TAIL, R_TAIL)])
            pltpu.sync_copy(deg_sh.at[pl.ds(N_U - R_TAIL, R_TAIL)],
                            ones_v.at[pl.ds(0, R_TAIL)])
            pltpu.sync_copy(ones_v.at[pl.ds(0, R_TAIL)],
                            deg_hbm.at[pl.ds(c * N_U + N_U - R_TAIL, R_TAIL)])

    return k(x_stack, src_all, dst_all)


def _tc_finish(aggs, degs, w_f, w_b):
    BR = 2000
    nblk = N_U // BR
    degs2 = degs.reshape(NC * N_U, 1)

    def body(a0_ref, a1_ref, d0_ref, d1_ref, w0_ref, w1_ref, o_ref):
        d0 = jnp.maximum(d0_ref[...], 1.0)
        d1 = jnp.maximum(d1_ref[...], 1.0)
        a0 = a0_ref[...] / d0
        a1 = a1_ref[...] / d1
        o_ref[...] = (
            jnp.dot(a0, w0_ref[...], preferred_element_type=jnp.float32)
            + jnp.dot(a1, w1_ref[...], preferred_element_type=jnp.float32)
        )

    return pl.pallas_call(
        body,
        grid=(nblk,),
        in_specs=[
            pl.BlockSpec((BR, D), lambda i: (i, 0)),
            pl.BlockSpec((BR, D), lambda i: (i + nblk, 0)),
            pl.BlockSpec((BR, 1), lambda i: (i, 0)),
            pl.BlockSpec((BR, 1), lambda i: (i + nblk, 0)),
            pl.BlockSpec((D, D), lambda i: (0, 0)),
            pl.BlockSpec((D, D), lambda i: (0, 0)),
        ],
        out_specs=pl.BlockSpec((BR, D), lambda i: (i, 0)),
        out_shape=jax.ShapeDtypeStruct((N_U, D), jnp.float32),
    )(aggs, aggs, degs2, degs2, w_f, w_b)


def kernel(x_user, x_item, edge_index_follows, edge_index_bought,
           W_follows, W_bought):
    src_f = edge_index_follows[0].astype(jnp.int32)
    dst_f = edge_index_follows[1].astype(jnp.int32)
    src_b = edge_index_bought[0].astype(jnp.int32) + N_U
    dst_b = edge_index_bought[1].astype(jnp.int32)
    x_stack = jnp.concatenate([x_user, x_item], axis=0)
    src_all = jnp.concatenate([src_f, src_b])
    dst_all = jnp.concatenate([dst_f, dst_b])
    aggs, degs = _sc_aggregate(x_stack, src_all, dst_all)
    return _tc_finish(aggs, degs, W_follows, W_bought)
